# baseline (speedup 1.0000x reference)
import functools

import jax
import jax.numpy as jnp
from jax import lax
from jax.experimental import pallas as pl
from jax.experimental.pallas import tpu as pltpu
from jax.experimental.pallas import tpu_sc as plsc

_NC = 2
_NS = 16
_NW = _NC * _NS
_LANES = 16
_BC = 128
_HB = 8


@functools.lru_cache(maxsize=None)
def _make_sc_gather(V, D, B, H):
    n_hb = H // _HB
    n_bc = B // _BC
    n_tasks = n_hb * n_bc
    per_w = n_tasks // _NW
    assert per_w * _NW == n_tasks
    scale = float(D) ** 0.5
    mesh = plsc.VectorSubcoreMesh(core_axis_name="c", subcore_axis_name="s")

    @functools.partial(
        pl.kernel,
        mesh=mesh,
        out_type=jax.ShapeDtypeStruct((H, D, B), jnp.float32),
        scratch_types=[
            pltpu.VMEM((_HB, _BC), jnp.int32),
            pltpu.VMEM((_BC,), jnp.int32),
            pltpu.VMEM((_HB, _LANES), jnp.int32),
            pltpu.VMEM((_BC, 2 * D), jnp.float32),
            pltpu.VMEM((D, _BC), jnp.float32),
            pltpu.SemaphoreType.DMA,
        ],
        compiler_params=pltpu.CompilerParams(
            use_tc_tiling_on_sc=True, needs_layout_passes=False),
    )
    def sc_kernel(xT_hbm, tab_hbm, out_hbm, idx_v, row_v, colb_v, rows_v,
                  tbuf_v, sem):
        wid = lax.axis_index("s") * _NC + lax.axis_index("c")

        @pl.loop(0, per_w)
        def _task(ti):
            t = wid * per_w + ti
            hb = t // n_bc
            bc = t % n_bc
            pltpu.sync_copy(
                xT_hbm.at[pl.ds(hb * _HB, _HB), pl.ds(bc * _BC, _BC)], idx_v)

            @pl.loop(0, _HB)
            def _chunk(hh):
                h = hb * _HB + hh
                for g in range(_BC // _LANES):
                    iv = idx_v[hh, pl.ds(g * _LANES, _LANES)]
                    row_v[pl.ds(g * _LANES, _LANES)] = (
                        lax.shift_right_logical(iv, 1))
                    colb_v[g, :] = (iv & 1) * D
                pltpu.async_copy(tab_hbm.at[row_v], rows_v, sem).wait()

                @pl.loop(0, D)
                def _f(f):
                    for g in range(_BC // _LANES):
                        ridx = lax.iota(jnp.int32, 16) + g * _LANES
                        cidx = colb_v[g, :] + f
                        vals = plsc.load_gather(rows_v, [ridx, cidx])
                        tbuf_v[f, pl.ds(g * _LANES, _LANES)] = vals * scale

                pltpu.sync_copy(tbuf_v, out_hbm.at[h, :, pl.ds(bc * _BC, _BC)])

    return sc_kernel


def kernel(x, table):
    B, H = x.shape
    V, D = table.shape
    sc = _make_sc_gather(V, D, B, H)
    xT = x.T
    tab = table.reshape(V // 2, 2 * D)
    outT = sc(xT, tab)
    return jnp.transpose(outT, (2, 0, 1))


# speedup vs baseline: 2.1599x; 2.1599x over previous
import functools

import jax
import jax.numpy as jnp
from jax import lax
from jax.experimental import pallas as pl
from jax.experimental.pallas import tpu as pltpu
from jax.experimental.pallas import tpu_sc as plsc

_NC = 2
_NS = 16
_NW = _NC * _NS
_LANES = 16
_BC = 128
_HB = 8


@functools.lru_cache(maxsize=None)
def _make_sc_gather(V, D, B, H):
    n_hb = H // _HB
    n_bc = B // _BC
    n_tasks = n_hb * n_bc
    per_w = n_tasks // _NW
    assert per_w * _NW == n_tasks
    n_g = _BC // _LANES
    scale = float(D) ** 0.5
    mesh = plsc.VectorSubcoreMesh(core_axis_name="c", subcore_axis_name="s")

    @functools.partial(
        pl.kernel,
        mesh=mesh,
        out_type=jax.ShapeDtypeStruct((H, D, B), jnp.float32),
        scratch_types=[
            pltpu.VMEM((_HB, _BC), jnp.int32),
            pltpu.VMEM((2, _BC), jnp.int32),
            pltpu.VMEM((2, n_g, _LANES), jnp.int32),
            pltpu.VMEM((2, _BC, 2 * D), jnp.float32),
            pltpu.VMEM((2, D, _BC), jnp.float32),
            pltpu.SemaphoreType.DMA((2,)),
            pltpu.SemaphoreType.DMA((2,)),
        ],
        compiler_params=pltpu.CompilerParams(
            use_tc_tiling_on_sc=True, needs_layout_passes=False),
    )
    def sc_kernel(xT_hbm, tab_hbm, out_hbm, idx_v, row_v, colb_v, rows_v,
                  tbuf_v, gsem, wsem):
        wid = lax.axis_index("s") * _NC + lax.axis_index("c")

        def prep(hh, s):
            for g in range(n_g):
                iv = idx_v[hh, pl.ds(g * _LANES, _LANES)]
                row_v[s, pl.ds(g * _LANES, _LANES)] = (
                    lax.shift_right_logical(iv, 1))
                colb_v[s, g, :] = (iv & 1) * D

        def gather_start(s):
            pltpu.async_copy(tab_hbm.at[row_v.at[s]], rows_v.at[s],
                             gsem.at[s])

        def gather_wait(s):
            pltpu.make_async_copy(tab_hbm.at[row_v.at[s]], rows_v.at[s],
                                  gsem.at[s]).wait()

        def write_start(h, bc, s):
            pltpu.async_copy(
                tbuf_v.at[s], out_hbm.at[h, :, pl.ds(bc * _BC, _BC)],
                wsem.at[s])

        def write_wait(h, bc, s):
            pltpu.make_async_copy(
                tbuf_v.at[s], out_hbm.at[h, :, pl.ds(bc * _BC, _BC)],
                wsem.at[s]).wait()

        @pl.loop(0, per_w)
        def _task(ti):
            t = wid * per_w + ti
            hb = t // n_bc
            bc = t % n_bc
            pltpu.sync_copy(
                xT_hbm.at[pl.ds(hb * _HB, _HB), pl.ds(bc * _BC, _BC)], idx_v)
            prep(0, 0)
            gather_start(0)
            for hh in range(_HB):
                s = hh % 2
                if hh + 1 < _HB:
                    prep(hh + 1, 1 - s)
                gather_wait(s)
                if hh + 1 < _HB:
                    gather_start(1 - s)
                if hh >= 2:
                    write_wait(0, 0, s)
                else:
                    @pl.when(ti > 0)
                    def _():
                        write_wait(0, 0, s)

                @plsc.parallel_loop(0, D, unroll=4)
                def _f(f):
                    for g in range(n_g):
                        ridx = lax.iota(jnp.int32, _LANES) + g * _LANES
                        cidx = colb_v[s, g, :] + f
                        vals = plsc.load_gather(rows_v.at[s], [ridx, cidx])
                        tbuf_v[s, f, pl.ds(g * _LANES, _LANES)] = vals * scale

                write_start(hb * _HB + hh, bc, s)

        write_wait(0, 0, 0)
        write_wait(0, 0, 1)

    return sc_kernel


def kernel(x, table):
    B, H = x.shape
    V, D = table.shape
    sc = _make_sc_gather(V, D, B, H)
    xT = x.T
    tab = table.reshape(V // 2, 2 * D)
    outT = sc(xT, tab)
    return jnp.transpose(outT, (2, 0, 1))


# speedup vs baseline: 2.2553x; 1.0442x over previous
import functools

import jax
import jax.numpy as jnp
from jax import lax
from jax.experimental import pallas as pl
from jax.experimental.pallas import tpu as pltpu
from jax.experimental.pallas import tpu_sc as plsc

_NC = 2
_NS = 16
_NW = _NC * _NS
_LANES = 16
_BC = 128
_HB = 8
_NGB = 4


@functools.lru_cache(maxsize=None)
def _make_sc_gather(V, D, B, H):
    n_hb = H // _HB
    n_bc = B // _BC
    n_tasks = n_hb * n_bc
    per_w = n_tasks // _NW
    assert per_w * _NW == n_tasks
    n_g = _BC // _LANES
    scale = float(D) ** 0.5
    mesh = plsc.VectorSubcoreMesh(core_axis_name="c", subcore_axis_name="s")

    @functools.partial(
        pl.kernel,
        mesh=mesh,
        out_type=jax.ShapeDtypeStruct((H, D, B), jnp.float32),
        scratch_types=[
            pltpu.VMEM((_HB, _BC), jnp.int32),
            pltpu.VMEM((_NGB, _BC), jnp.int32),
            pltpu.VMEM((_NGB, n_g, _LANES), jnp.int32),
            pltpu.VMEM((_NGB, _BC, 2 * D), jnp.float32),
            pltpu.VMEM((2, D, _BC), jnp.float32),
            pltpu.SemaphoreType.DMA((_NGB,)),
            pltpu.SemaphoreType.DMA((2,)),
        ],
        compiler_params=pltpu.CompilerParams(
            use_tc_tiling_on_sc=True, needs_layout_passes=False),
    )
    def sc_kernel(xT_hbm, tab_hbm, out_hbm, idx_v, row_v, colb_v, rows_v,
                  tbuf_v, gsem, wsem):
        wid = lax.axis_index("s") * _NC + lax.axis_index("c")

        def prep(hh):
            s = hh % _NGB
            for g in range(n_g):
                iv = idx_v[hh, pl.ds(g * _LANES, _LANES)]
                row_v[s, pl.ds(g * _LANES, _LANES)] = (
                    lax.shift_right_logical(iv, 1))
                colb_v[s, g, :] = (iv & 1) * D

        def gather_start(s):
            pltpu.async_copy(tab_hbm.at[row_v.at[s]], rows_v.at[s],
                             gsem.at[s])

        def gather_wait(s):
            pltpu.make_async_copy(tab_hbm.at[row_v.at[s]], rows_v.at[s],
                                  gsem.at[s]).wait()

        def write_start(h, bc, s):
            pltpu.async_copy(
                tbuf_v.at[s], out_hbm.at[h, :, pl.ds(bc * _BC, _BC)],
                wsem.at[s])

        def write_wait(s):
            pltpu.make_async_copy(
                tbuf_v.at[s], out_hbm.at[0, :, pl.ds(0, _BC)],
                wsem.at[s]).wait()

        @pl.loop(0, per_w)
        def _task(ti):
            t = wid * per_w + ti
            hb = t // n_bc
            bc = t % n_bc
            pltpu.sync_copy(
                xT_hbm.at[pl.ds(hb * _HB, _HB), pl.ds(bc * _BC, _BC)], idx_v)
            for hh in range(_NGB - 1):
                prep(hh)
                gather_start(hh)
            for hh in range(_HB):
                s = hh % _NGB
                ws = hh % 2
                gather_wait(s)
                if hh + _NGB - 1 < _HB:
                    prep(hh + _NGB - 1)
                    gather_start((hh + _NGB - 1) % _NGB)
                if hh >= 2:
                    write_wait(ws)
                else:
                    @pl.when(ti > 0)
                    def _():
                        write_wait(ws)

                ridx = [lax.iota(jnp.int32, _LANES) + g * _LANES
                        for g in range(n_g)]
                colb = [colb_v[s, g, :] for g in range(n_g)]

                @plsc.parallel_loop(0, D, unroll=8)
                def _f(f):
                    for g in range(n_g):
                        vals = plsc.load_gather(rows_v.at[s],
                                                [ridx[g], colb[g] + f])
                        tbuf_v[ws, f, pl.ds(g * _LANES, _LANES)] = vals * scale

                write_start(hb * _HB + hh, bc, ws)

        write_wait(0)
        write_wait(1)

    return sc_kernel


def kernel(x, table):
    B, H = x.shape
    V, D = table.shape
    sc = _make_sc_gather(V, D, B, H)
    xT = x.T
    tab = table.reshape(V // 2, 2 * D)
    outT = sc(xT, tab)
    return jnp.transpose(outT, (2, 0, 1))


# speedup vs baseline: 2.4420x; 1.0828x over previous
import functools

import jax
import jax.numpy as jnp
from jax import lax
from jax.experimental import pallas as pl
from jax.experimental.pallas import tpu as pltpu
from jax.experimental.pallas import tpu_sc as plsc

_NC = 2
_NS = 16
_NW = _NC * _NS
_LANES = 16
_CHUNK = 128
_NBUF = 5


@functools.lru_cache(maxsize=None)
def _make_sc_gather(V, D, TOT):
    per_w = TOT // _NW
    n_chunks = per_w // _CHUNK
    assert n_chunks % _NBUF == 0
    scale = float(D) ** 0.5
    mesh = plsc.VectorSubcoreMesh(core_axis_name="c", subcore_axis_name="s")

    @functools.partial(
        pl.kernel,
        mesh=mesh,
        out_type=jax.ShapeDtypeStruct((TOT, D), jnp.float32),
        scratch_types=[
            pltpu.VMEM((n_chunks, _CHUNK), jnp.int32),
            pltpu.VMEM((_NBUF, _CHUNK, D), jnp.float32),
            pltpu.SemaphoreType.DMA((_NBUF,)),
            pltpu.SemaphoreType.DMA((_NBUF,)),
        ],
        compiler_params=pltpu.CompilerParams(use_tc_tiling_on_sc=False),
    )
    def sc_kernel(x_hbm, table_hbm, out_hbm, idx_v, rows_v, gsem, wsem):
        wid = lax.axis_index("s") * _NC + lax.axis_index("c")
        base = wid * per_w
        pltpu.sync_copy(x_hbm.at[wid], idx_v)

        def gather_start(j, b):
            pltpu.async_copy(table_hbm.at[idx_v.at[j]], rows_v.at[b], gsem.at[b])

        def gather_wait(j, b):
            pltpu.make_async_copy(
                table_hbm.at[idx_v.at[j]], rows_v.at[b], gsem.at[b]).wait()

        def write_start(j, b):
            pltpu.async_copy(
                rows_v.at[b], out_hbm.at[pl.ds(base + j * _CHUNK, _CHUNK)],
                wsem.at[b])

        def write_wait(b):
            pltpu.make_async_copy(
                rows_v.at[b], out_hbm.at[pl.ds(base, _CHUNK)], wsem.at[b]).wait()

        for b in range(_NBUF - 1):
            gather_start(b, b)

        @pl.loop(0, n_chunks, step=_NBUF)
        def _group(j0):
            for b in range(_NBUF):
                j = j0 + b
                b2 = (b + _NBUF - 1) % _NBUF
                jn = j + _NBUF - 1

                @pl.when(jn < n_chunks)
                def _():
                    @pl.when(j > 0)
                    def _():
                        write_wait(b2)
                    gather_start(jn, b2)

                gather_wait(j, b)

                @plsc.parallel_loop(0, _CHUNK, unroll=4)
                def _row(r):
                    for c in range(D // _LANES):
                        sl = pl.ds(c * _LANES, _LANES)
                        rows_v[b, r, sl] = rows_v[b, r, sl] * scale

                write_start(j, b)

        for b in range(_NBUF):
            write_wait(b)

    return sc_kernel


def kernel(x, table):
    B, H = x.shape
    V, D = table.shape
    TOT = B * H
    sc = _make_sc_gather(V, D, TOT)
    xr = x.reshape(_NW, TOT // _NW // _CHUNK, _CHUNK).astype(jnp.int32)
    out = sc(xr, table)
    return out.reshape(B, H, D)
